# baseline (speedup 1.0000x reference)
import functools

import jax
import jax.numpy as jnp
from jax import lax
from jax.experimental import pallas as pl
from jax.experimental.pallas import tpu as pltpu
from jax.experimental.pallas import tpu_sc as plsc

L = 16
N_ROWS = 128
N_COLS = 32768
NC = 2
NS = 16
NW = NC * NS
ROWS_PER_W = N_ROWS // NW
N_CHUNKS = N_COLS // L
UNROLL = 8

_INT_MAX = 0x7FFFFFFF


def _row_absmax_value(buf):
    def body(j, carry):
        best_a, best_i, best_v, cur_i = carry
        for u in range(UNROLL):
            v = buf[pl.ds((j * UNROLL + u) * L, L)]
            a = jnp.abs(v)
            pred = a > best_a
            best_a = jnp.where(pred, a, best_a)
            best_i = jnp.where(pred, cur_i, best_i)
            best_v = jnp.where(pred, v, best_v)
            cur_i = cur_i + L
        return best_a, best_i, best_v, cur_i

    init = (
        jnp.full((L,), -1.0, jnp.float32),
        jnp.zeros((L,), jnp.int32),
        jnp.zeros((L,), jnp.float32),
        lax.iota(jnp.int32, L),
    )
    best_a, best_i, best_v, _ = lax.fori_loop(0, N_CHUNKS // UNROLL, body, init)

    lane = lax.iota(jnp.int32, L)
    for shift in (1, 2, 4, 8):
        perm = lane ^ shift
        other_a = best_a.at[perm].get(mode="promise_in_bounds")
        other_i = best_i.at[perm].get(mode="promise_in_bounds")
        other_v = best_v.at[perm].get(mode="promise_in_bounds")
        pred = (other_a > best_a) | ((other_a == best_a) & (other_i < best_i))
        best_a = jnp.where(pred, other_a, best_a)
        best_i = jnp.where(pred, other_i, best_i)
        best_v = jnp.where(pred, other_v, best_v)
    return best_v


def _sc_body(neuron_hbm, seg_hbm, out_hbm, buf0, buf1, nvec, ovec, sem0, sem1):
    cid = lax.axis_index("c")
    sid = lax.axis_index("s")
    wid = sid * NC + cid
    base = wid * ROWS_PER_W

    pltpu.sync_copy(neuron_hbm.at[wid], nvec)

    bufs = (buf0, buf1)
    sems = (sem0, sem1)
    first = pltpu.make_async_copy(seg_hbm.at[base], buf0, sem0)
    first.start()
    pending = [first, None]

    lane = lax.iota(jnp.int32, L)
    percent = jnp.zeros((L,), jnp.float32)
    for r in range(ROWS_PER_W):
        cur = r % 2
        if r + 1 < ROWS_PER_W:
            nxt = (r + 1) % 2
            nxt_copy = pltpu.make_async_copy(
                seg_hbm.at[base + r + 1], bufs[nxt], sems[nxt])
            nxt_copy.start()
            pending[nxt] = nxt_copy
        pending[cur].wait()
        g = _row_absmax_value(bufs[cur])
        p = 1.0 / (1.0 + jnp.exp(-g))
        percent = jnp.where(lane == r, p, percent)

    ovec[...] = nvec[...] * percent
    pltpu.sync_copy(ovec, out_hbm.at[wid])


_sc_call = functools.partial(
    pl.kernel,
    mesh=plsc.VectorSubcoreMesh(core_axis_name="c", subcore_axis_name="s"),
    out_type=jax.ShapeDtypeStruct((NW, L), jnp.float32),
    scratch_types=[
        pltpu.VMEM((N_COLS,), jnp.float32),
        pltpu.VMEM((N_COLS,), jnp.float32),
        pltpu.VMEM((L,), jnp.float32),
        pltpu.VMEM((L,), jnp.float32),
        pltpu.SemaphoreType.DMA,
        pltpu.SemaphoreType.DMA,
    ],
)(_sc_body)


@jax.jit
def kernel(neuron_out, segment_out):
    neuron_tiled = jnp.zeros((NW, L), jnp.float32)
    neuron_tiled = neuron_tiled.at[:, :ROWS_PER_W].set(
        neuron_out.reshape(NW, ROWS_PER_W))
    out = _sc_call(neuron_tiled, segment_out)
    return out[:, :ROWS_PER_W].reshape(N_ROWS)
